# baseline (speedup 1.0000x reference)
import functools

import jax
import jax.numpy as jnp
from jax import lax
from jax.experimental import pallas as pl
from jax.experimental.pallas import tpu as pltpu
from jax.experimental.pallas import tpu_sc as plsc

N_USERS = 50000
EMB = 128
B = 1024
L = 32

NW = 32
ROWS = B * L
ROWS_PER_W = ROWS // NW
CH = 128
N_CHUNKS = ROWS_PER_W // CH


@functools.cache
def _make_gather():
    mesh = plsc.VectorSubcoreMesh(core_axis_name="c", subcore_axis_name="s")

    @functools.partial(
        pl.kernel,
        mesh=mesh,
        out_type=[
            jax.ShapeDtypeStruct((ROWS, EMB), jnp.float32),
            jax.ShapeDtypeStruct((ROWS, EMB), jnp.float32),
        ],
        scratch_types=[
            pltpu.VMEM((N_CHUNKS, CH), jnp.int32),
            pltpu.VMEM((2, CH, EMB), jnp.float32),
            pltpu.VMEM((2, CH, EMB), jnp.float32),
        ] + [pltpu.SemaphoreType.DMA] * 8,
    )
    def gather_k(emb_hbm, prof_hbm, idx_hbm, out_e, out_p,
                 idx_v, buf_e, buf_p, *sems):
        gsem_e, gsem_p, wsem_e, wsem_p = sems[0:2], sems[2:4], sems[4:6], sems[6:8]
        wid = lax.axis_index("s") * 2 + lax.axis_index("c")
        pltpu.sync_copy(idx_hbm.at[wid], idx_v)
        base = wid * ROWS_PER_W

        gathers = {}
        writes = {}

        def issue_gather(c):
            b = c % 2
            gathers[c] = (
                pltpu.async_copy(emb_hbm.at[idx_v.at[c]], buf_e.at[b], gsem_e[b]),
                pltpu.async_copy(prof_hbm.at[idx_v.at[c]], buf_p.at[b], gsem_p[b]),
            )

        def issue_write(c):
            b = c % 2
            row0 = base + c * CH
            writes[c] = (
                pltpu.async_copy(buf_e.at[b], out_e.at[pl.ds(row0, CH)], wsem_e[b]),
                pltpu.async_copy(buf_p.at[b], out_p.at[pl.ds(row0, CH)], wsem_p[b]),
            )

        issue_gather(0)
        for c in range(N_CHUNKS):
            if c + 1 < N_CHUNKS:
                if c - 1 >= 0:
                    for w in writes[c - 1]:
                        w.wait()
                issue_gather(c + 1)
            for g in gathers[c]:
                g.wait()
            issue_write(c)
        for c in (N_CHUNKS - 2, N_CHUNKS - 1):
            for w in writes[c]:
                w.wait()

    return gather_k


def _mlp_body(ge_ref, gp_ref, i_ref, u_ref, wf_ref, bf_ref, wc1_ref, bc1_ref,
              wc2_ref, bc2_ref, comb_ref, att_ref):
    BB = i_ref.shape[0]
    bf16 = jnp.bfloat16
    x = jnp.concatenate([ge_ref[...], gp_ref[...]], axis=-1).astype(bf16)
    h0 = jnp.dot(x, wf_ref[...].astype(bf16),
                 preferred_element_type=jnp.float32)
    h0 = jnp.maximum(h0 + bf_ref[...], 0.0)

    wc1a = wc1_ref[0:EMB, :].astype(bf16)
    wc1b = wc1_ref[EMB:2 * EMB, :].astype(bf16)
    iterm = jnp.dot(i_ref[...].astype(bf16), wc1b,
                    preferred_element_type=jnp.float32)
    iterm = iterm + bc1_ref[...]
    iterm3 = jnp.broadcast_to(iterm[:, None, :], (BB, L, EMB))
    c1 = jnp.dot(h0.astype(bf16), wc1a, preferred_element_type=jnp.float32)
    c1 = jnp.maximum(c1 + iterm3.reshape(BB * L, EMB), 0.0)

    c2 = jnp.dot(c1.astype(bf16), wc2_ref[...].astype(bf16),
                 preferred_element_type=jnp.float32)
    c2 = jnp.maximum(c2 + bc2_ref[...], 0.0)

    c2_3d = c2.reshape(BB, L, EMB)
    u3 = jnp.broadcast_to(u_ref[...][:, None, :], (BB, L, EMB))
    scores = jnp.sum(c2_3d * u3, axis=-1) * (1.0 / (EMB ** 0.5))
    m = jnp.max(scores, axis=-1, keepdims=True)
    e = jnp.exp(scores - m)
    att = e / jnp.sum(e, axis=-1, keepdims=True)
    att_ref[...] = att
    comb_ref[...] = jnp.sum(c2_3d * att[:, :, None], axis=1)


def _mlp_att(ge, gp, i_embs, u_embs, W_f, b_f, W_c1, b_c1, W_c2, b_c2):
    BB = 128
    grid = (B // BB,)
    full = lambda i: (0, 0)
    blk = lambda i: (i, 0)
    return pl.pallas_call(
        _mlp_body,
        grid=grid,
        in_specs=[
            pl.BlockSpec((BB * L, EMB), blk),
            pl.BlockSpec((BB * L, EMB), blk),
            pl.BlockSpec((BB, EMB), blk),
            pl.BlockSpec((BB, EMB), blk),
            pl.BlockSpec((2 * EMB, EMB), full),
            pl.BlockSpec((1, EMB), full),
            pl.BlockSpec((2 * EMB, EMB), full),
            pl.BlockSpec((1, EMB), full),
            pl.BlockSpec((EMB, EMB), full),
            pl.BlockSpec((1, EMB), full),
        ],
        out_specs=[
            pl.BlockSpec((BB, EMB), blk),
            pl.BlockSpec((BB, L), blk),
        ],
        out_shape=[
            jax.ShapeDtypeStruct((B, EMB), jnp.float32),
            jax.ShapeDtypeStruct((B, L), jnp.float32),
        ],
    )(ge, gp, i_embs, u_embs, W_f, b_f, W_c1, b_c1, W_c2, b_c2)


def kernel(users, u_embs, items, i_embs, act_users, user_embs_weight,
           user_profiles, W_f, b_f, W_c1, b_c1, W_c2, b_c2):
    idx = act_users.astype(jnp.int32).reshape(NW, N_CHUNKS, CH)
    ge, gp = _make_gather()(user_embs_weight, user_profiles, idx)
    comb, att = _mlp_att(ge, gp, i_embs, u_embs, W_f,
                         b_f.reshape(1, EMB), W_c1, b_c1.reshape(1, EMB),
                         W_c2, b_c2.reshape(1, EMB))
    return comb, att[..., None]


# speedup vs baseline: 1.0113x; 1.0113x over previous
import functools

import jax
import jax.numpy as jnp
from jax import lax
from jax.experimental import pallas as pl
from jax.experimental.pallas import tpu as pltpu
from jax.experimental.pallas import tpu_sc as plsc

N_USERS = 50000
EMB = 128
B = 1024
L = 32

NW = 32
ROWS = B * L
ROWS_PER_W = ROWS // NW
CH = 128
N_CHUNKS = ROWS_PER_W // CH


@functools.cache
def _make_gather():
    mesh = plsc.VectorSubcoreMesh(core_axis_name="c", subcore_axis_name="s")

    @functools.partial(
        pl.kernel,
        mesh=mesh,
        out_type=[
            jax.ShapeDtypeStruct((ROWS, EMB), jnp.float32),
            jax.ShapeDtypeStruct((ROWS, EMB), jnp.float32),
        ],
        scratch_types=[
            pltpu.VMEM((N_CHUNKS, CH), jnp.int32),
            pltpu.VMEM((3, CH, EMB), jnp.float32),
            pltpu.VMEM((3, CH, EMB), jnp.float32),
        ] + [pltpu.SemaphoreType.DMA] * 12,
    )
    def gather_k(emb_hbm, prof_hbm, idx_hbm, out_e, out_p,
                 idx_v, buf_e, buf_p, *sems):
        gsem_e, gsem_p, wsem_e, wsem_p = sems[0:3], sems[3:6], sems[6:9], sems[9:12]
        wid = lax.axis_index("s") * 2 + lax.axis_index("c")
        pltpu.sync_copy(idx_hbm.at[wid], idx_v)
        base = wid * ROWS_PER_W

        gathers = {}
        writes = {}

        def issue_gather(c):
            b = c % 3
            gathers[c] = (
                pltpu.async_copy(emb_hbm.at[idx_v.at[c]], buf_e.at[b], gsem_e[b]),
                pltpu.async_copy(prof_hbm.at[idx_v.at[c]], buf_p.at[b], gsem_p[b]),
            )

        def issue_write(c):
            b = c % 3
            row0 = base + c * CH
            writes[c] = (
                pltpu.async_copy(buf_e.at[b], out_e.at[pl.ds(row0, CH)], wsem_e[b]),
                pltpu.async_copy(buf_p.at[b], out_p.at[pl.ds(row0, CH)], wsem_p[b]),
            )

        issue_gather(0)
        for c in range(N_CHUNKS):
            if c + 1 < N_CHUNKS:
                if c - 2 >= 0:
                    for w in writes[c - 2]:
                        w.wait()
                issue_gather(c + 1)
            for g in gathers[c]:
                g.wait()
            issue_write(c)
        for c in (N_CHUNKS - 3, N_CHUNKS - 2, N_CHUNKS - 1):
            for w in writes[c]:
                w.wait()

    return gather_k


def _mlp_body(ge_ref, gp_ref, i_ref, u_ref, wf_ref, bf_ref, wc1_ref, bc1_ref,
              wc2_ref, bc2_ref, comb_ref, att_ref):
    BB = i_ref.shape[0]
    bf16 = jnp.bfloat16
    x = jnp.concatenate([ge_ref[...], gp_ref[...]], axis=-1).astype(bf16)
    h0 = jnp.dot(x, wf_ref[...].astype(bf16),
                 preferred_element_type=jnp.float32)
    h0 = jnp.maximum(h0 + bf_ref[...], 0.0)

    wc1a = wc1_ref[0:EMB, :].astype(bf16)
    wc1b = wc1_ref[EMB:2 * EMB, :].astype(bf16)
    iterm = jnp.dot(i_ref[...].astype(bf16), wc1b,
                    preferred_element_type=jnp.float32)
    iterm = iterm + bc1_ref[...]
    iterm3 = jnp.broadcast_to(iterm[:, None, :], (BB, L, EMB))
    c1 = jnp.dot(h0.astype(bf16), wc1a, preferred_element_type=jnp.float32)
    c1 = jnp.maximum(c1 + iterm3.reshape(BB * L, EMB), 0.0)

    c2 = jnp.dot(c1.astype(bf16), wc2_ref[...].astype(bf16),
                 preferred_element_type=jnp.float32)
    c2 = jnp.maximum(c2 + bc2_ref[...], 0.0)

    c2_3d = c2.reshape(BB, L, EMB)
    u3 = jnp.broadcast_to(u_ref[...][:, None, :], (BB, L, EMB))
    scores = jnp.sum(c2_3d * u3, axis=-1) * (1.0 / (EMB ** 0.5))
    m = jnp.max(scores, axis=-1, keepdims=True)
    e = jnp.exp(scores - m)
    att = e / jnp.sum(e, axis=-1, keepdims=True)
    att_ref[...] = att
    comb_ref[...] = jnp.sum(c2_3d * att[:, :, None], axis=1)


def _mlp_att(ge, gp, i_embs, u_embs, W_f, b_f, W_c1, b_c1, W_c2, b_c2):
    BB = 128
    grid = (B // BB,)
    full = lambda i: (0, 0)
    blk = lambda i: (i, 0)
    return pl.pallas_call(
        _mlp_body,
        grid=grid,
        in_specs=[
            pl.BlockSpec((BB * L, EMB), blk),
            pl.BlockSpec((BB * L, EMB), blk),
            pl.BlockSpec((BB, EMB), blk),
            pl.BlockSpec((BB, EMB), blk),
            pl.BlockSpec((2 * EMB, EMB), full),
            pl.BlockSpec((1, EMB), full),
            pl.BlockSpec((2 * EMB, EMB), full),
            pl.BlockSpec((1, EMB), full),
            pl.BlockSpec((EMB, EMB), full),
            pl.BlockSpec((1, EMB), full),
        ],
        out_specs=[
            pl.BlockSpec((BB, EMB), blk),
            pl.BlockSpec((BB, L), blk),
        ],
        out_shape=[
            jax.ShapeDtypeStruct((B, EMB), jnp.float32),
            jax.ShapeDtypeStruct((B, L), jnp.float32),
        ],
    )(ge, gp, i_embs, u_embs, W_f, b_f, W_c1, b_c1, W_c2, b_c2)


def kernel(users, u_embs, items, i_embs, act_users, user_embs_weight,
           user_profiles, W_f, b_f, W_c1, b_c1, W_c2, b_c2):
    idx = act_users.astype(jnp.int32).reshape(NW, N_CHUNKS, CH)
    ge, gp = _make_gather()(user_embs_weight, user_profiles, idx)
    comb, att = _mlp_att(ge, gp, i_embs, u_embs, W_f,
                         b_f.reshape(1, EMB), W_c1, b_c1.reshape(1, EMB),
                         W_c2, b_c2.reshape(1, EMB))
    return comb, att[..., None]


# speedup vs baseline: 1.0269x; 1.0154x over previous
import functools

import jax
import jax.numpy as jnp
from jax import lax
from jax.experimental import pallas as pl
from jax.experimental.pallas import tpu as pltpu
from jax.experimental.pallas import tpu_sc as plsc

N_USERS = 50000
EMB = 128
B = 1024
L = 32

NW = 32
ROWS = B * L
CH = 128
N_CHUNKS = ROWS // NW // CH


@functools.cache
def _make_gather(n_chunks):
    rows = NW * n_chunks * CH
    rows_per_w = n_chunks * CH
    mesh = plsc.VectorSubcoreMesh(core_axis_name="c", subcore_axis_name="s")

    @functools.partial(
        pl.kernel,
        mesh=mesh,
        out_type=[
            jax.ShapeDtypeStruct((rows, EMB), jnp.float32),
            jax.ShapeDtypeStruct((rows, EMB), jnp.float32),
        ],
        scratch_types=[
            pltpu.VMEM((n_chunks, CH), jnp.int32),
            pltpu.VMEM((3, CH, EMB), jnp.float32),
            pltpu.VMEM((3, CH, EMB), jnp.float32),
        ] + [pltpu.SemaphoreType.DMA] * 12,
    )
    def gather_k(emb_hbm, prof_hbm, idx_hbm, out_e, out_p,
                 idx_v, buf_e, buf_p, *sems):
        gsem_e, gsem_p, wsem_e, wsem_p = sems[0:3], sems[3:6], sems[6:9], sems[9:12]
        wid = lax.axis_index("s") * 2 + lax.axis_index("c")
        pltpu.sync_copy(idx_hbm.at[wid], idx_v)
        base = wid * rows_per_w

        gathers = {}
        writes = {}

        def issue_gather(c):
            b = c % 3
            gathers[c] = (
                pltpu.async_copy(emb_hbm.at[idx_v.at[c]], buf_e.at[b], gsem_e[b]),
                pltpu.async_copy(prof_hbm.at[idx_v.at[c]], buf_p.at[b], gsem_p[b]),
            )

        def issue_write(c):
            b = c % 3
            row0 = base + c * CH
            writes[c] = (
                pltpu.async_copy(buf_e.at[b], out_e.at[pl.ds(row0, CH)], wsem_e[b]),
                pltpu.async_copy(buf_p.at[b], out_p.at[pl.ds(row0, CH)], wsem_p[b]),
            )

        issue_gather(0)
        for c in range(n_chunks):
            if c + 1 < n_chunks:
                if c - 2 >= 0:
                    for w in writes[c - 2]:
                        w.wait()
                issue_gather(c + 1)
            for g in gathers[c]:
                g.wait()
            issue_write(c)
        for c in range(max(0, n_chunks - 3), n_chunks):
            for w in writes[c]:
                w.wait()

    return gather_k


def _mlp_body(ge_ref, gp_ref, i_ref, u_ref, wf_ref, bf_ref, wc1_ref, bc1_ref,
              wc2_ref, bc2_ref, comb_ref, att_ref):
    BB = i_ref.shape[0]
    bf16 = jnp.bfloat16
    x = jnp.concatenate([ge_ref[...], gp_ref[...]], axis=-1).astype(bf16)
    h0 = jnp.dot(x, wf_ref[...].astype(bf16),
                 preferred_element_type=jnp.float32)
    h0 = jnp.maximum(h0 + bf_ref[...], 0.0)

    wc1a = wc1_ref[0:EMB, :].astype(bf16)
    wc1b = wc1_ref[EMB:2 * EMB, :].astype(bf16)
    iterm = jnp.dot(i_ref[...].astype(bf16), wc1b,
                    preferred_element_type=jnp.float32)
    iterm = iterm + bc1_ref[...]
    iterm3 = jnp.broadcast_to(iterm[:, None, :], (BB, L, EMB))
    c1 = jnp.dot(h0.astype(bf16), wc1a, preferred_element_type=jnp.float32)
    c1 = jnp.maximum(c1 + iterm3.reshape(BB * L, EMB), 0.0)

    c2 = jnp.dot(c1.astype(bf16), wc2_ref[...].astype(bf16),
                 preferred_element_type=jnp.float32)
    c2 = jnp.maximum(c2 + bc2_ref[...], 0.0)

    c2_3d = c2.reshape(BB, L, EMB)
    u3 = jnp.broadcast_to(u_ref[...][:, None, :], (BB, L, EMB))
    scores = jnp.sum(c2_3d * u3, axis=-1) * (1.0 / (EMB ** 0.5))
    m = jnp.max(scores, axis=-1, keepdims=True)
    e = jnp.exp(scores - m)
    att = e / jnp.sum(e, axis=-1, keepdims=True)
    att_ref[...] = att
    comb_ref[...] = jnp.sum(c2_3d * att[:, :, None], axis=1)


def _mlp_att(ge, gp, i_embs, u_embs, W_f, b_f, W_c1, b_c1, W_c2, b_c2):
    bh = i_embs.shape[0]
    BB = 128
    grid = (bh // BB,)
    full = lambda i: (0, 0)
    blk = lambda i: (i, 0)
    return pl.pallas_call(
        _mlp_body,
        grid=grid,
        in_specs=[
            pl.BlockSpec((BB * L, EMB), blk),
            pl.BlockSpec((BB * L, EMB), blk),
            pl.BlockSpec((BB, EMB), blk),
            pl.BlockSpec((BB, EMB), blk),
            pl.BlockSpec((2 * EMB, EMB), full),
            pl.BlockSpec((1, EMB), full),
            pl.BlockSpec((2 * EMB, EMB), full),
            pl.BlockSpec((1, EMB), full),
            pl.BlockSpec((EMB, EMB), full),
            pl.BlockSpec((1, EMB), full),
        ],
        out_specs=[
            pl.BlockSpec((BB, EMB), blk),
            pl.BlockSpec((BB, L), blk),
        ],
        out_shape=[
            jax.ShapeDtypeStruct((bh, EMB), jnp.float32),
            jax.ShapeDtypeStruct((bh, L), jnp.float32),
        ],
    )(ge, gp, i_embs, u_embs, W_f, b_f, W_c1, b_c1, W_c2, b_c2)


H = 2


def kernel(users, u_embs, items, i_embs, act_users, user_embs_weight,
           user_profiles, W_f, b_f, W_c1, b_c1, W_c2, b_c2):
    nc = N_CHUNKS // H
    bh = B // H
    idx = act_users.astype(jnp.int32).reshape(H, NW, nc, CH)
    gather = _make_gather(nc)
    gathered = [gather(user_embs_weight, user_profiles, idx[h])
                for h in range(H)]
    bf = b_f.reshape(1, EMB)
    bc1 = b_c1.reshape(1, EMB)
    bc2 = b_c2.reshape(1, EMB)
    outs = [
        _mlp_att(ge, gp, i_embs[h * bh:(h + 1) * bh],
                 u_embs[h * bh:(h + 1) * bh],
                 W_f, bf, W_c1, bc1, W_c2, bc2)
        for h, (ge, gp) in enumerate(gathered)
    ]
    comb = jnp.concatenate([c for c, _ in outs])
    att = jnp.concatenate([a for _, a in outs])
    return comb, att[..., None]


# speedup vs baseline: 1.1224x; 1.0930x over previous
import functools

import jax
import jax.numpy as jnp
from jax import lax
from jax.experimental import pallas as pl
from jax.experimental.pallas import tpu as pltpu
from jax.experimental.pallas import tpu_sc as plsc

N_USERS = 50000
EMB = 128
B = 1024
L = 32

NW = 32
ROWS = B * L
CH = 128
N_CHUNKS = ROWS // NW // CH


@functools.cache
def _make_gather(n_chunks):
    rows = NW * n_chunks * CH
    rows_per_w = n_chunks * CH
    mesh = plsc.VectorSubcoreMesh(core_axis_name="c", subcore_axis_name="s")

    @functools.partial(
        pl.kernel,
        mesh=mesh,
        out_type=[
            jax.ShapeDtypeStruct((rows, EMB), jnp.float32),
            jax.ShapeDtypeStruct((rows, EMB), jnp.float32),
        ],
        scratch_types=[
            pltpu.VMEM((n_chunks, CH), jnp.int32),
            pltpu.VMEM((3, CH, EMB), jnp.float32),
            pltpu.VMEM((3, CH, EMB), jnp.float32),
        ] + [pltpu.SemaphoreType.DMA] * 12,
    )
    def gather_k(emb_hbm, prof_hbm, idx_hbm, out_e, out_p,
                 idx_v, buf_e, buf_p, *sems):
        gsem_e, gsem_p, wsem_e, wsem_p = sems[0:3], sems[3:6], sems[6:9], sems[9:12]
        wid = lax.axis_index("s") * 2 + lax.axis_index("c")
        pltpu.sync_copy(idx_hbm.at[wid], idx_v)
        base = wid * rows_per_w

        gathers = {}
        writes = {}

        def issue_gather(c):
            b = c % 3
            gathers[c] = (
                pltpu.async_copy(emb_hbm.at[idx_v.at[c]], buf_e.at[b], gsem_e[b]),
                pltpu.async_copy(prof_hbm.at[idx_v.at[c]], buf_p.at[b], gsem_p[b]),
            )

        def issue_write(c):
            b = c % 3
            row0 = base + c * CH
            writes[c] = (
                pltpu.async_copy(buf_e.at[b], out_e.at[pl.ds(row0, CH)], wsem_e[b]),
                pltpu.async_copy(buf_p.at[b], out_p.at[pl.ds(row0, CH)], wsem_p[b]),
            )

        issue_gather(0)
        for c in range(n_chunks):
            if c + 1 < n_chunks:
                if c - 2 >= 0:
                    for w in writes[c - 2]:
                        w.wait()
                issue_gather(c + 1)
            for g in gathers[c]:
                g.wait()
            issue_write(c)
        for c in range(max(0, n_chunks - 3), n_chunks):
            for w in writes[c]:
                w.wait()

    return gather_k


def _mlp_body(ge_ref, gp_ref, i_ref, u_ref, wf_ref, bf_ref, wc1a_ref,
              wc1b_ref, bc1_ref, wc2_ref, bc2_ref, ones_ref,
              comb_ref, att_ref):
    BB = i_ref.shape[0]
    bf16 = jnp.bfloat16
    x = jnp.concatenate([ge_ref[...], gp_ref[...]], axis=-1).astype(bf16)
    h0 = jnp.dot(x, wf_ref[...], preferred_element_type=jnp.float32)
    h0 = jnp.maximum(h0 + bf_ref[...], 0.0)

    iterm = jnp.dot(i_ref[...].astype(bf16), wc1b_ref[...],
                    preferred_element_type=jnp.float32)
    iterm = iterm + bc1_ref[...]
    iterm3 = jnp.broadcast_to(iterm[:, None, :], (BB, L, EMB))
    c1 = jnp.dot(h0.astype(bf16), wc1a_ref[...],
                 preferred_element_type=jnp.float32)
    c1 = jnp.maximum(c1 + iterm3.reshape(BB * L, EMB), 0.0)

    c2 = jnp.dot(c1.astype(bf16), wc2_ref[...],
                 preferred_element_type=jnp.float32)
    c2 = jnp.maximum(c2 + bc2_ref[...], 0.0)

    us = u_ref[...] * (1.0 / (EMB ** 0.5))
    u3 = jnp.broadcast_to(us[:, None, :], (BB, L, EMB)).reshape(BB * L, EMB)
    v = (c2 * u3).astype(bf16)
    s_b = jnp.dot(v, ones_ref[...], preferred_element_type=jnp.float32)
    s3 = s_b.reshape(BB, L, EMB)
    m = jnp.max(s3, axis=1, keepdims=True)
    e = jnp.exp(s3 - m)
    den = jnp.sum(e, axis=1, keepdims=True)
    att3 = e / den
    comb_ref[...] = jnp.sum(c2.reshape(BB, L, EMB) * att3, axis=1)
    att_ref[...] = att3[:, :, 0]


def _mlp_att(ge, gp, i_embs, u_embs, wf, b_f, wc1a, wc1b, b_c1, wc2, b_c2,
             ones_b):
    bh = i_embs.shape[0]
    BB = 128
    grid = (bh // BB,)
    full = lambda i: (0, 0)
    blk = lambda i: (i, 0)
    return pl.pallas_call(
        _mlp_body,
        grid=grid,
        in_specs=[
            pl.BlockSpec((BB * L, EMB), blk),
            pl.BlockSpec((BB * L, EMB), blk),
            pl.BlockSpec((BB, EMB), blk),
            pl.BlockSpec((BB, EMB), blk),
            pl.BlockSpec((2 * EMB, EMB), full),
            pl.BlockSpec((1, EMB), full),
            pl.BlockSpec((EMB, EMB), full),
            pl.BlockSpec((EMB, EMB), full),
            pl.BlockSpec((1, EMB), full),
            pl.BlockSpec((EMB, EMB), full),
            pl.BlockSpec((1, EMB), full),
            pl.BlockSpec((EMB, EMB), full),
        ],
        out_specs=[
            pl.BlockSpec((BB, EMB), blk),
            pl.BlockSpec((BB, L), blk),
        ],
        out_shape=[
            jax.ShapeDtypeStruct((bh, EMB), jnp.float32),
            jax.ShapeDtypeStruct((bh, L), jnp.float32),
        ],
    )(ge, gp, i_embs, u_embs, wf, b_f, wc1a, wc1b, b_c1, wc2, b_c2, ones_b)


H = 2


def kernel(users, u_embs, items, i_embs, act_users, user_embs_weight,
           user_profiles, W_f, b_f, W_c1, b_c1, W_c2, b_c2):
    nc = N_CHUNKS // H
    bh = B // H
    idx = act_users.astype(jnp.int32).reshape(H, NW, nc, CH)
    gather = _make_gather(nc)
    gathered = [gather(user_embs_weight, user_profiles, idx[h])
                for h in range(H)]
    bf16 = jnp.bfloat16
    wf = W_f.astype(bf16)
    wc1a = W_c1[:EMB].astype(bf16)
    wc1b = W_c1[EMB:].astype(bf16)
    wc2 = W_c2.astype(bf16)
    ones_b = jnp.ones((EMB, EMB), bf16)
    bf = b_f.reshape(1, EMB)
    bc1 = b_c1.reshape(1, EMB)
    bc2 = b_c2.reshape(1, EMB)
    outs = [
        _mlp_att(ge, gp, i_embs[h * bh:(h + 1) * bh],
                 u_embs[h * bh:(h + 1) * bh],
                 wf, bf, wc1a, wc1b, bc1, wc2, bc2, ones_b)
        for h, (ge, gp) in enumerate(gathered)
    ]
    comb = jnp.concatenate([c for c, _ in outs])
    att = jnp.concatenate([a for _, a in outs])
    return comb, att[..., None]


# speedup vs baseline: 1.1282x; 1.0052x over previous
import functools

import jax
import jax.numpy as jnp
from jax import lax
from jax.experimental import pallas as pl
from jax.experimental.pallas import tpu as pltpu
from jax.experimental.pallas import tpu_sc as plsc

N_USERS = 50000
EMB = 128
B = 1024
L = 32

NW = 32
ROWS = B * L
CH = 128
N_CHUNKS = ROWS // NW // CH


@functools.cache
def _make_gather(n_chunks):
    rows = NW * n_chunks * CH
    rows_per_w = n_chunks * CH
    mesh = plsc.VectorSubcoreMesh(core_axis_name="c", subcore_axis_name="s")

    @functools.partial(
        pl.kernel,
        mesh=mesh,
        out_type=[
            jax.ShapeDtypeStruct((rows, EMB), jnp.float32),
            jax.ShapeDtypeStruct((rows, EMB), jnp.float32),
        ],
        scratch_types=[
            pltpu.VMEM((n_chunks, CH), jnp.int32),
            pltpu.VMEM((3, CH, EMB), jnp.float32),
            pltpu.VMEM((3, CH, EMB), jnp.float32),
        ] + [pltpu.SemaphoreType.DMA] * 12,
    )
    def gather_k(emb_hbm, prof_hbm, idx_hbm, out_e, out_p,
                 idx_v, buf_e, buf_p, *sems):
        gsem_e, gsem_p, wsem_e, wsem_p = sems[0:3], sems[3:6], sems[6:9], sems[9:12]
        wid = lax.axis_index("s") * 2 + lax.axis_index("c")
        pltpu.sync_copy(idx_hbm.at[wid], idx_v)
        base = wid * rows_per_w

        gathers = {}
        writes = {}

        def issue_gather(c):
            b = c % 3
            gathers[c] = (
                pltpu.async_copy(emb_hbm.at[idx_v.at[c]], buf_e.at[b], gsem_e[b]),
                pltpu.async_copy(prof_hbm.at[idx_v.at[c]], buf_p.at[b], gsem_p[b]),
            )

        def issue_write(c):
            b = c % 3
            row0 = base + c * CH
            writes[c] = (
                pltpu.async_copy(buf_e.at[b], out_e.at[pl.ds(row0, CH)], wsem_e[b]),
                pltpu.async_copy(buf_p.at[b], out_p.at[pl.ds(row0, CH)], wsem_p[b]),
            )

        issue_gather(0)
        for c in range(n_chunks):
            if c + 1 < n_chunks:
                if c - 2 >= 0:
                    for w in writes[c - 2]:
                        w.wait()
                issue_gather(c + 1)
            for g in gathers[c]:
                g.wait()
            issue_write(c)
        for c in range(max(0, n_chunks - 3), n_chunks):
            for w in writes[c]:
                w.wait()

    return gather_k


def _mlp_body(ge_ref, gp_ref, i_ref, u_ref, wf_ref, bf_ref, wc1a_ref,
              wc1b_ref, bc1_ref, wc2_ref, bc2_ref, ones_ref,
              comb_ref, att_ref):
    BB = i_ref.shape[0]
    bf16 = jnp.bfloat16
    x = jnp.concatenate([ge_ref[...], gp_ref[...]], axis=-1).astype(bf16)
    h0 = jnp.dot(x, wf_ref[...], preferred_element_type=jnp.float32)
    h0 = jnp.maximum(h0 + bf_ref[...], 0.0)

    iterm = jnp.dot(i_ref[...].astype(bf16), wc1b_ref[...],
                    preferred_element_type=jnp.float32)
    iterm = iterm + bc1_ref[...]
    iterm3 = jnp.broadcast_to(iterm[:, None, :], (BB, L, EMB))
    c1 = jnp.dot(h0.astype(bf16), wc1a_ref[...],
                 preferred_element_type=jnp.float32)
    c1 = jnp.maximum(c1 + iterm3.reshape(BB * L, EMB), 0.0)

    c2 = jnp.dot(c1.astype(bf16), wc2_ref[...],
                 preferred_element_type=jnp.float32)
    c2 = jnp.maximum(c2 + bc2_ref[...], 0.0)

    us = u_ref[...] * (1.0 / (EMB ** 0.5))
    u3 = jnp.broadcast_to(us[:, None, :], (BB, L, EMB)).reshape(BB * L, EMB)
    v = (c2 * u3).astype(bf16)
    s_b = jnp.dot(v, ones_ref[...], preferred_element_type=jnp.float32)
    s3 = s_b.reshape(BB, L, EMB)
    m = jnp.max(s3, axis=1, keepdims=True)
    e = jnp.exp(s3 - m)
    den = jnp.sum(e, axis=1, keepdims=True)
    att3 = e / den
    comb_ref[...] = jnp.sum(c2.reshape(BB, L, EMB) * att3, axis=1)
    att_ref[...] = att3[:, :, 0]


def _mlp_att(ge, gp, i_embs, u_embs, wf, b_f, wc1a, wc1b, b_c1, wc2, b_c2,
             ones_b):
    bh = i_embs.shape[0]
    BB = 256
    grid = (bh // BB,)
    full = lambda i: (0, 0)
    blk = lambda i: (i, 0)
    return pl.pallas_call(
        _mlp_body,
        grid=grid,
        in_specs=[
            pl.BlockSpec((BB * L, EMB), blk),
            pl.BlockSpec((BB * L, EMB), blk),
            pl.BlockSpec((BB, EMB), blk),
            pl.BlockSpec((BB, EMB), blk),
            pl.BlockSpec((2 * EMB, EMB), full),
            pl.BlockSpec((1, EMB), full),
            pl.BlockSpec((EMB, EMB), full),
            pl.BlockSpec((EMB, EMB), full),
            pl.BlockSpec((1, EMB), full),
            pl.BlockSpec((EMB, EMB), full),
            pl.BlockSpec((1, EMB), full),
            pl.BlockSpec((EMB, EMB), full),
        ],
        out_specs=[
            pl.BlockSpec((BB, EMB), blk),
            pl.BlockSpec((BB, L), blk),
        ],
        out_shape=[
            jax.ShapeDtypeStruct((bh, EMB), jnp.float32),
            jax.ShapeDtypeStruct((bh, L), jnp.float32),
        ],
    )(ge, gp, i_embs, u_embs, wf, b_f, wc1a, wc1b, b_c1, wc2, b_c2, ones_b)


H = 2


def kernel(users, u_embs, items, i_embs, act_users, user_embs_weight,
           user_profiles, W_f, b_f, W_c1, b_c1, W_c2, b_c2):
    nc = N_CHUNKS // H
    bh = B // H
    idx = act_users.astype(jnp.int32).reshape(H, NW, nc, CH)
    gather = _make_gather(nc)
    gathered = [gather(user_embs_weight, user_profiles, idx[h])
                for h in range(H)]
    bf16 = jnp.bfloat16
    wf = W_f.astype(bf16)
    wc1a = W_c1[:EMB].astype(bf16)
    wc1b = W_c1[EMB:].astype(bf16)
    wc2 = W_c2.astype(bf16)
    ones_b = jnp.ones((EMB, EMB), bf16)
    bf = b_f.reshape(1, EMB)
    bc1 = b_c1.reshape(1, EMB)
    bc2 = b_c2.reshape(1, EMB)
    outs = [
        _mlp_att(ge, gp, i_embs[h * bh:(h + 1) * bh],
                 u_embs[h * bh:(h + 1) * bh],
                 wf, bf, wc1a, wc1b, bc1, wc2, bc2, ones_b)
        for h, (ge, gp) in enumerate(gathered)
    ]
    comb = jnp.concatenate([c for c, _ in outs])
    att = jnp.concatenate([a for _, a in outs])
    return comb, att[..., None]
